# baseline (speedup 1.0000x reference)
import functools

import jax
import jax.numpy as jnp
from jax import lax
from jax.experimental import pallas as pl
from jax.experimental.pallas import tpu as pltpu
from jax.experimental.pallas import tpu_sc as plsc

_N = 16777216
_NC = 2
_NS = 16
_NW = _NC * _NS
_PER_W = _N // _NW
_CHUNK = 16384
_NCHUNK = _PER_W // _CHUNK
_NBUF = 4

_mesh = plsc.VectorSubcoreMesh(core_axis_name="c", subcore_axis_name="s")


def _compute(buf):
    @plsc.parallel_loop(0, _CHUNK, 16, unroll=16)
    def vec_body(vi):
        v = buf[pl.ds(vi, 16)]
        buf[pl.ds(vi, 16)] = jnp.where(v > 1.0, 1.0, 0.0)


@functools.partial(
    pl.kernel,
    mesh=_mesh,
    out_type=jax.ShapeDtypeStruct((_N,), jnp.float32),
    scratch_types=(
        [pltpu.VMEM((_CHUNK,), jnp.float32)] * _NBUF
        + [pltpu.SemaphoreType.DMA] * (2 * _NBUF)
    ),
)
def _sc_binarize(x_hbm, o_hbm, *scratch):
    bufs = scratch[:_NBUF]
    gsems = scratch[_NBUF:2 * _NBUF]
    ssems = scratch[2 * _NBUF:]
    wid = lax.axis_index("s") * _NC + lax.axis_index("c")
    base = wid * _PER_W

    def gather_start(ci):
        b = ci % _NBUF
        pltpu.make_async_copy(
            x_hbm.at[pl.ds(base + ci * _CHUNK, _CHUNK)], bufs[b], gsems[b]
        ).start()

    def gather_wait(ci):
        b = ci % _NBUF
        pltpu.make_async_copy(
            x_hbm.at[pl.ds(base + ci * _CHUNK, _CHUNK)], bufs[b], gsems[b]
        ).wait()

    def scatter_start(ci):
        b = ci % _NBUF
        pltpu.make_async_copy(
            bufs[b], o_hbm.at[pl.ds(base + ci * _CHUNK, _CHUNK)], ssems[b]
        ).start()

    def scatter_wait(ci):
        b = ci % _NBUF
        pltpu.make_async_copy(
            bufs[b], o_hbm.at[pl.ds(base + ci * _CHUNK, _CHUNK)], ssems[b]
        ).wait()

    for ci in range(_NBUF - 1):
        gather_start(ci)
    for ci in range(_NCHUNK):
        gather_wait(ci)
        _compute(bufs[ci % _NBUF])
        scatter_start(ci)
        if ci + _NBUF - 1 < _NCHUNK:
            if ci >= 1:
                scatter_wait(ci - 1)
            gather_start(ci + _NBUF - 1)
    for ci in range(_NCHUNK - _NBUF, _NCHUNK):
        scatter_wait(ci)


def kernel(x):
    return _sc_binarize(x)


# speedup vs baseline: 1.0326x; 1.0326x over previous
import functools

import jax
import jax.numpy as jnp
from jax import lax
from jax.experimental import pallas as pl
from jax.experimental.pallas import tpu as pltpu
from jax.experimental.pallas import tpu_sc as plsc

_N = 16777216
_NC = 2
_NS = 16
_NW = _NC * _NS
_PER_W = _N // _NW
_CHUNK = 16384
_NCHUNK = _PER_W // _CHUNK
_VPC = _CHUNK // 16

_mesh = plsc.VectorSubcoreMesh(core_axis_name="c", subcore_axis_name="s")


def _compute(src, dst):
    @plsc.parallel_loop(0, _CHUNK, 16, unroll=8)
    def vec_body(vi):
        v = src[pl.ds(vi, 16)]
        dst[pl.ds(vi, 16)] = jnp.where(v > 1.0, 1.0, 0.0)


@functools.partial(
    pl.kernel,
    mesh=_mesh,
    out_type=jax.ShapeDtypeStruct((_N,), jnp.float32),
    scratch_types=[
        pltpu.VMEM((_CHUNK,), jnp.float32),
        pltpu.VMEM((_CHUNK,), jnp.float32),
        pltpu.VMEM((_CHUNK,), jnp.float32),
        pltpu.VMEM((_CHUNK,), jnp.float32),
        pltpu.SemaphoreType.DMA,
        pltpu.SemaphoreType.DMA,
        pltpu.SemaphoreType.DMA,
        pltpu.SemaphoreType.DMA,
    ],
)
def _sc_binarize(x_hbm, o_hbm, in0, in1, out0, out1, gs0, gs1, ss0, ss1):
    slots = ((in0, out0, gs0, ss0), (in1, out1, gs1, ss1))
    wid = lax.axis_index("s") * _NC + lax.axis_index("c")
    base = wid * _PER_W

    def gather(ci, ib, gs):
        pltpu.make_async_copy(
            x_hbm.at[pl.ds(base + ci * _CHUNK, _CHUNK)], ib, gs).start()

    def gather_wait(ci, ib, gs):
        pltpu.make_async_copy(
            x_hbm.at[pl.ds(base + ci * _CHUNK, _CHUNK)], ib, gs).wait()

    def scatter(ci, ob, ss):
        pltpu.make_async_copy(
            ob, o_hbm.at[pl.ds(base + ci * _CHUNK, _CHUNK)], ss).start()

    def scatter_wait(ci, ob, ss):
        pltpu.make_async_copy(
            ob, o_hbm.at[pl.ds(base + ci * _CHUNK, _CHUNK)], ss).wait()

    gather(0, in0, gs0)
    gather(1, in1, gs1)

    for b in range(2):
        ib, ob, gs, ss = slots[b]
        gather_wait(b, ib, gs)
        _compute(ib, ob)
        scatter(b, ob, ss)
        gather(b + 2, ib, gs)

    def group_body(g, carry):
        for b in range(2):
            ib, ob, gs, ss = slots[b]
            ci = 2 * g + b
            gather_wait(ci, ib, gs)
            scatter_wait(ci - 2, ob, ss)
            _compute(ib, ob)
            scatter(ci, ob, ss)
            gather(ci + 2, ib, gs)
        return carry

    lax.fori_loop(1, _NCHUNK // 2 - 1, group_body, 0)

    for b in range(2):
        ib, ob, gs, ss = slots[b]
        ci = _NCHUNK - 2 + b
        gather_wait(ci, ib, gs)
        scatter_wait(ci - 2, ob, ss)
        _compute(ib, ob)
        scatter(ci, ob, ss)
    for b in range(2):
        ib, ob, gs, ss = slots[b]
        scatter_wait(_NCHUNK - 2 + b, ob, ss)


def kernel(x):
    return _sc_binarize(x)
